# baseline (speedup 1.0000x reference)
import functools

import jax
import jax.numpy as jnp
from jax import lax
from jax.experimental import pallas as pl
from jax.experimental.pallas import tpu as pltpu
from jax.experimental.pallas import tpu_sc as plsc

N_NODES = 10000
N_EDGES = 320000
D_IN = 128
D_HID = 128
N_CLASSES = 40

NC = 2
NS = 16
LANES = 16
NPAIR = D_HID // 2
PPW = NPAIR // NS
E_HALF = N_EDGES // NC
CH = 1280
NCH = E_HALF // CH
GR = CH // LANES
UNROLL = 3
MASKHI = jnp.int32(-65536)

_SC_MESH = plsc.VectorSubcoreMesh(
    core_axis_name="c", subcore_axis_name="s", num_cores=NC, num_subcores=NS)


@functools.partial(
    pl.kernel,
    out_type=jax.ShapeDtypeStruct((NC * D_HID * N_NODES,), jnp.float32),
    mesh=_SC_MESH,
    compiler_params=pltpu.CompilerParams(needs_layout_passes=False),
    scratch_types=[
        pltpu.VMEM((PPW * N_NODES,), jnp.int32),
        pltpu.VMEM((2 * PPW * N_NODES,), jnp.float32),
        pltpu.VMEM((2, CH), jnp.int32),
        pltpu.VMEM((2, CH), jnp.int32),
        pltpu.VMEM((2, CH), jnp.float32),
        pltpu.SemaphoreType.DMA,
        pltpu.SemaphoreType.DMA,
    ],
)
def _spmm_sc(tab_hbm, row_hbm, col_hbm, val_hbm, out_hbm,
             tab_v, acc_v, rowb, colb, valb, sem0, sem1):
    c = lax.axis_index("c")
    s = lax.axis_index("s")
    sems = (sem0, sem1)
    ebase = c * E_HALF

    def _copies(ci, b):
        off = ebase + ci * CH
        return (
            pltpu.make_async_copy(row_hbm.at[pl.ds(off, CH)], rowb.at[b], sems[b]),
            pltpu.make_async_copy(col_hbm.at[pl.ds(off, CH)], colb.at[b], sems[b]),
            pltpu.make_async_copy(val_hbm.at[pl.ds(off, CH)], valb.at[b], sems[b]),
        )

    for cp in _copies(0, 0):
        cp.start()

    pltpu.sync_copy(tab_hbm.at[pl.ds(s * (PPW * N_NODES), PPW * N_NODES)], tab_v)

    zero = jnp.zeros((LANES,), jnp.float32)

    @plsc.parallel_loop(0, 2 * PPW * N_NODES // LANES, unroll=8)
    def _zbody(i):
        acc_v[pl.ds(i * LANES, LANES)] = zero

    def _process(b):
        @plsc.parallel_loop(0, GR, unroll=UNROLL)
        def _gbody(g):
            sl = pl.ds(g * LANES, LANES)
            row = rowb[b, sl]
            col = colb[b, sl]
            val = valb[b, sl]
            for p in range(PPW):
                gidx = col if p == 0 else col + (p * N_NODES)
                packed = plsc.load_gather(tab_v, [gidx])
                fa = plsc.bitcast(packed << 16, jnp.float32)
                fb = plsc.bitcast(packed & MASKHI, jnp.float32)
                sa = row if p == 0 else row + (p * N_NODES)
                sb = row + ((PPW + p) * N_NODES)
                plsc.addupdate_scatter(acc_v, [sa], fa * val)
                plsc.addupdate_scatter(acc_v, [sb], fb * val)

    def _pair(i, carry):
        for b in range(2):
            ci = i * 2 + b

            @pl.when(ci + 1 < NCH)
            def _():
                for cp in _copies(ci + 1, 1 - b):
                    cp.start()

            for cp in _copies(ci, b):
                cp.wait()
            _process(b)
        return carry

    lax.fori_loop(0, NCH // 2, _pair, 0)

    if NCH % 2 == 1:
        for cp in _copies(NCH - 1, 0):
            cp.wait()
        _process(0)

    cbase = c * (D_HID * N_NODES)
    half = PPW * N_NODES
    pltpu.sync_copy(acc_v.at[pl.ds(0, half)],
                    out_hbm.at[pl.ds(cbase + s * half, half)])
    pltpu.sync_copy(acc_v.at[pl.ds(half, half)],
                    out_hbm.at[pl.ds(cbase + NPAIR * N_NODES + s * half, half)])


def _pack_pairs(sT):
    u = lax.bitcast_convert_type(sT, jnp.uint32)
    a = u[:NPAIR]
    b = u[NPAIR:]
    ra = (a + jnp.uint32(0x7FFF) + ((a >> 16) & jnp.uint32(1))) >> 16
    rb = (b + jnp.uint32(0x7FFF) + ((b >> 16) & jnp.uint32(1))) & jnp.uint32(0xFFFF0000)
    return lax.bitcast_convert_type(ra | rb, jnp.int32)


def _mm1_body(x_ref, w_ref, o_ref):
    sT = lax.dot_general(
        w_ref[...], x_ref[...], (((0,), (1,)), ((), ())),
        preferred_element_type=jnp.float32)
    o_ref[...] = _pack_pairs(sT)


def _dense1(x, W):
    return pl.pallas_call(
        _mm1_body,
        out_shape=jax.ShapeDtypeStruct((NPAIR, N_NODES), jnp.int32),
    )(x, W)


def _mm2_body(p_ref, b_ref, w_ref, o_ref):
    h = jnp.maximum(p_ref[0] + p_ref[1] + b_ref[...], 0.0)
    sT = lax.dot_general(
        w_ref[...], h, (((0,), (0,)), ((), ())),
        preferred_element_type=jnp.float32)
    o_ref[...] = _pack_pairs(sT)


def _dense2(p, b, W):
    return pl.pallas_call(
        _mm2_body,
        out_shape=jax.ShapeDtypeStruct((NPAIR, N_NODES), jnp.int32),
    )(p, b, W)


def _mm3_body(p_ref, b_ref, w_ref, cb_ref, o_ref):
    h2 = p_ref[0] + p_ref[1] + b_ref[...]
    logits = lax.dot_general(
        h2, w_ref[...], (((0,), (0,)), ((), ())),
        preferred_element_type=jnp.float32) + cb_ref[...]
    m = jnp.max(logits, axis=-1, keepdims=True)
    lse = jnp.log(jnp.sum(jnp.exp(logits - m), axis=-1, keepdims=True)) + m
    o_ref[...] = logits - lse


def _dense3(p, b, W, cb):
    return pl.pallas_call(
        _mm3_body,
        out_shape=jax.ShapeDtypeStruct((N_NODES, N_CLASSES), jnp.float32),
    )(p, b, W, cb)


def kernel(x, edge_index, adj_values, gc1_W, gc1_b, gc2_W, gc2_b, cls_W, cls_b):
    row = edge_index[0].astype(jnp.int32)
    col = edge_index[1].astype(jnp.int32)
    val = adj_values.astype(jnp.float32)
    b1 = gc1_b.reshape(D_HID, 1)
    b2 = gc2_b.reshape(D_HID, 1)
    cb = cls_b.reshape(1, N_CLASSES)

    pk1 = _dense1(x, gc1_W)
    pp1 = _spmm_sc(pk1.reshape(-1), row, col, val).reshape(NC, D_HID, N_NODES)
    pk2 = _dense2(pp1, b1, gc2_W)
    pp2 = _spmm_sc(pk2.reshape(-1), row, col, val).reshape(NC, D_HID, N_NODES)
    return _dense3(pp2, b2, cls_W, cb)


# speedup vs baseline: 1.0346x; 1.0346x over previous
import functools

import jax
import jax.numpy as jnp
from jax import lax
from jax.experimental import pallas as pl
from jax.experimental.pallas import tpu as pltpu
from jax.experimental.pallas import tpu_sc as plsc

N_NODES = 10000
N_EDGES = 320000
D_IN = 128
D_HID = 128
N_CLASSES = 40

NC = 2
NS = 16
LANES = 16
NPAIR = D_HID // 2
PPW = NPAIR // NS
E_HALF = N_EDGES // NC
CH = 1280
NCH = E_HALF // CH
GR = CH // LANES
UNROLL = 2
MASKHI = jnp.int32(-65536)

_SC_MESH = plsc.VectorSubcoreMesh(
    core_axis_name="c", subcore_axis_name="s", num_cores=NC, num_subcores=NS)


@functools.partial(
    pl.kernel,
    out_type=jax.ShapeDtypeStruct((NC * D_HID * N_NODES,), jnp.float32),
    mesh=_SC_MESH,
    compiler_params=pltpu.CompilerParams(needs_layout_passes=False),
    scratch_types=[
        pltpu.VMEM((PPW * N_NODES,), jnp.int32),
        pltpu.VMEM((2 * PPW * N_NODES,), jnp.float32),
        pltpu.VMEM((2, CH), jnp.int32),
        pltpu.VMEM((2, CH), jnp.int32),
        pltpu.VMEM((2, CH), jnp.float32),
        pltpu.SemaphoreType.DMA,
        pltpu.SemaphoreType.DMA,
    ],
)
def _spmm_sc(tab_hbm, row_hbm, col_hbm, val_hbm, out_hbm,
             tab_v, acc_v, rowb, colb, valb, sem0, sem1):
    c = lax.axis_index("c")
    s = lax.axis_index("s")
    sems = (sem0, sem1)
    ebase = c * E_HALF

    def _copies(ci, b):
        off = ebase + ci * CH
        return (
            pltpu.make_async_copy(row_hbm.at[pl.ds(off, CH)], rowb.at[b], sems[b]),
            pltpu.make_async_copy(col_hbm.at[pl.ds(off, CH)], colb.at[b], sems[b]),
            pltpu.make_async_copy(val_hbm.at[pl.ds(off, CH)], valb.at[b], sems[b]),
        )

    for cp in _copies(0, 0):
        cp.start()

    pltpu.sync_copy(tab_hbm.at[pl.ds(s * (PPW * N_NODES), PPW * N_NODES)], tab_v)

    zero = jnp.zeros((LANES,), jnp.float32)

    @plsc.parallel_loop(0, 2 * PPW * N_NODES // LANES, unroll=8)
    def _zbody(i):
        acc_v[pl.ds(i * LANES, LANES)] = zero

    def _process(b):
        @plsc.parallel_loop(0, GR, unroll=UNROLL)
        def _gbody(g):
            sl = pl.ds(g * LANES, LANES)
            row = rowb[b, sl]
            col = colb[b, sl]
            val = valb[b, sl]
            for p in range(PPW):
                gidx = col if p == 0 else col + (p * N_NODES)
                packed = plsc.load_gather(tab_v, [gidx])
                fa = plsc.bitcast(packed << 16, jnp.float32)
                fb = plsc.bitcast(packed & MASKHI, jnp.float32)
                sa = row if p == 0 else row + (p * N_NODES)
                sb = row + ((PPW + p) * N_NODES)
                plsc.addupdate_scatter(acc_v, [sa], fa * val)
                plsc.addupdate_scatter(acc_v, [sb], fb * val)

    def _pair(i, carry):
        for b in range(2):
            ci = i * 2 + b

            @pl.when(ci + 1 < NCH)
            def _():
                for cp in _copies(ci + 1, 1 - b):
                    cp.start()

            for cp in _copies(ci, b):
                cp.wait()
            _process(b)
        return carry

    lax.fori_loop(0, NCH // 2, _pair, 0)

    if NCH % 2 == 1:
        for cp in _copies(NCH - 1, 0):
            cp.wait()
        _process(0)

    cbase = c * (D_HID * N_NODES)
    half = PPW * N_NODES
    pltpu.sync_copy(acc_v.at[pl.ds(0, half)],
                    out_hbm.at[pl.ds(cbase + s * half, half)])
    pltpu.sync_copy(acc_v.at[pl.ds(half, half)],
                    out_hbm.at[pl.ds(cbase + NPAIR * N_NODES + s * half, half)])


def _pack_pairs(sT):
    u = lax.bitcast_convert_type(sT, jnp.uint32)
    a = u[:NPAIR]
    b = u[NPAIR:]
    ra = (a + jnp.uint32(0x7FFF) + ((a >> 16) & jnp.uint32(1))) >> 16
    rb = (b + jnp.uint32(0x7FFF) + ((b >> 16) & jnp.uint32(1))) & jnp.uint32(0xFFFF0000)
    return lax.bitcast_convert_type(ra | rb, jnp.int32)


def _mm1_body(x_ref, w_ref, o_ref):
    sT = lax.dot_general(
        w_ref[...], x_ref[...], (((0,), (1,)), ((), ())),
        preferred_element_type=jnp.float32)
    o_ref[...] = _pack_pairs(sT)


def _dense1(x, W):
    return pl.pallas_call(
        _mm1_body,
        out_shape=jax.ShapeDtypeStruct((NPAIR, N_NODES), jnp.int32),
    )(x, W)


def _mm2_body(p_ref, b_ref, w_ref, o_ref):
    h = jnp.maximum(p_ref[0] + p_ref[1] + b_ref[...], 0.0)
    sT = lax.dot_general(
        w_ref[...], h, (((0,), (0,)), ((), ())),
        preferred_element_type=jnp.float32)
    o_ref[...] = _pack_pairs(sT)


def _dense2(p, b, W):
    return pl.pallas_call(
        _mm2_body,
        out_shape=jax.ShapeDtypeStruct((NPAIR, N_NODES), jnp.int32),
    )(p, b, W)


def _mm3_body(p_ref, b_ref, w_ref, cb_ref, o_ref):
    h2 = p_ref[0] + p_ref[1] + b_ref[...]
    logits = lax.dot_general(
        h2, w_ref[...], (((0,), (0,)), ((), ())),
        preferred_element_type=jnp.float32) + cb_ref[...]
    m = jnp.max(logits, axis=-1, keepdims=True)
    lse = jnp.log(jnp.sum(jnp.exp(logits - m), axis=-1, keepdims=True)) + m
    o_ref[...] = logits - lse


def _dense3(p, b, W, cb):
    return pl.pallas_call(
        _mm3_body,
        out_shape=jax.ShapeDtypeStruct((N_NODES, N_CLASSES), jnp.float32),
    )(p, b, W, cb)


def kernel(x, edge_index, adj_values, gc1_W, gc1_b, gc2_W, gc2_b, cls_W, cls_b):
    row = edge_index[0].astype(jnp.int32)
    col = edge_index[1].astype(jnp.int32)
    val = adj_values.astype(jnp.float32)
    b1 = gc1_b.reshape(D_HID, 1)
    b2 = gc2_b.reshape(D_HID, 1)
    cb = cls_b.reshape(1, N_CLASSES)

    pk1 = _dense1(x, gc1_W)
    pp1 = _spmm_sc(pk1.reshape(-1), row, col, val).reshape(NC, D_HID, N_NODES)
    pk2 = _dense2(pp1, b1, gc2_W)
    pp2 = _spmm_sc(pk2.reshape(-1), row, col, val).reshape(NC, D_HID, N_NODES)
    return _dense3(pp2, b2, cls_W, cb)
